# baseline (speedup 1.0000x reference)
import functools

import jax
import jax.numpy as jnp
from jax import lax
from jax.experimental import pallas as pl
from jax.experimental.pallas import tpu as pltpu
from jax.experimental.pallas import tpu_sc as plsc

N = 10000
D = 128
E = 160000

NPAD = 10240
CHUNK = 128
G = 4
NGRP = 20
NCH = NGRP * G
EPT = NCH * CHUNK
NEP = 16 * EPT
RPT = NPAD // 16
ZCH = RPT // CHUNK
LPC = CHUNK // 16


def _sc_aggregate(table, src_all, dst_all):
    mesh = plsc.VectorSubcoreMesh(core_axis_name="c", subcore_axis_name="s")

    @functools.partial(
        pl.kernel,
        mesh=mesh,
        compiler_params=pltpu.CompilerParams(use_tc_tiling_on_sc=False,
                                             needs_layout_passes=False),
        out_type=(jax.ShapeDtypeStruct((2 * NPAD, D), jnp.float32),
                  jax.ShapeDtypeStruct((2, NPAD), jnp.float32)),
        scratch_types=[
            [pltpu.VMEM((G, CHUNK), jnp.int32) for _ in range(2)],
            [pltpu.VMEM((G, CHUNK), jnp.int32) for _ in range(2)],
            [pltpu.VMEM((CHUNK, D), jnp.float32) for _ in range(2)],
            pltpu.VMEM((NPAD,), jnp.float32),
            pltpu.VMEM_SHARED((NPAD, D), jnp.float32),
            pltpu.HBM((2, 16, NPAD), jnp.float32),
            pltpu.SemaphoreType.DMA,
            [pltpu.SemaphoreType.DMA for _ in range(2)],
            [pltpu.SemaphoreType.DMA for _ in range(2)],
        ],
    )
    def sc_agg(table_hbm, src_hbm, dst_hbm, out_hbm, deg_hbm, src_g, dst_g,
               rows, deg_v, acc_sh, part_hbm, gsem, ssem, isem):
        cid = lax.axis_index("c")
        sid = lax.axis_index("s")
        widx = cid * 16 + sid
        row0 = sid * RPT

        zeros16 = jnp.zeros((16,), jnp.float32)
        ones16 = jnp.ones((16,), jnp.float32)

        def zero_row(i, carry):
            for c in range(D // 16):
                rows[0][i, pl.ds(c * 16, 16)] = zeros16
            return carry

        lax.fori_loop(0, CHUNK, zero_row, 0)

        def zero_deg(i, carry):
            deg_v[pl.ds(i * 16, 16)] = zeros16
            return carry

        lax.fori_loop(0, NPAD // 16, zero_deg, 0)
        for j in range(ZCH):
            pltpu.sync_copy(rows[0], acc_sh.at[pl.ds(row0 + j * CHUNK, CHUNK)])
        plsc.subcore_barrier()

        def chunk(sg, dg, k, drain):
            b = k % 2
            if drain:
                pltpu.make_async_copy(table_hbm.at[pl.ds(0, CHUNK)], rows[b],
                                      ssem[b]).wait()
            pltpu.async_copy(table_hbm.at[sg.at[k]], rows[b], gsem).wait()
            pltpu.async_copy(rows[b], acc_sh.at[dg.at[k]], ssem[b], add=True)
            for l in range(LPC):
                dvec = dg[k, pl.ds(l * 16, 16)]
                plsc.addupdate_scatter(deg_v, [dvec], ones16)

        def stage(g, par):
            pltpu.async_copy(src_hbm.at[widx, g], src_g[par], isem[par])
            pltpu.async_copy(dst_hbm.at[widx, g], dst_g[par], isem[par])

        def stage_wait(par):
            pltpu.make_async_copy(src_hbm.at[widx, 0], src_g[par],
                                  isem[par]).wait()
            pltpu.make_async_copy(dst_hbm.at[widx, 0], dst_g[par],
                                  isem[par]).wait()

        pltpu.sync_copy(src_hbm.at[widx, 0], src_g[0])
        pltpu.sync_copy(dst_hbm.at[widx, 0], dst_g[0])
        stage(1, 1)
        chunk(src_g[0], dst_g[0], 0, False)
        chunk(src_g[0], dst_g[0], 1, False)
        chunk(src_g[0], dst_g[0], 2, True)
        chunk(src_g[0], dst_g[0], 3, True)
        stage(2, 0)

        def pair_body(p, carry):
            g1 = p * 2 + 1
            stage_wait(1)
            for k in range(G):
                chunk(src_g[1], dst_g[1], k, True)
            stage(g1 + 2, 1)
            stage_wait(0)
            for k in range(G):
                chunk(src_g[0], dst_g[0], k, True)
            stage(jnp.minimum(g1 + 3, NGRP - 1), 0)
            return carry

        lax.fori_loop(0, (NGRP - 2) // 2, pair_body, 0)

        stage_wait(1)
        for k in range(G):
            chunk(src_g[1], dst_g[1], k, True)
        pltpu.make_async_copy(table_hbm.at[pl.ds(0, CHUNK)], rows[0],
                              ssem[0]).wait()
        pltpu.make_async_copy(table_hbm.at[pl.ds(0, CHUNK)], rows[1],
                              ssem[1]).wait()
        stage_wait(0)

        pltpu.sync_copy(deg_v, part_hbm.at[cid, sid])
        plsc.subcore_barrier()
        for t in range(16):
            pltpu.sync_copy(part_hbm.at[cid, t, pl.ds(row0, RPT)],
                            deg_v.at[pl.ds(t * RPT, RPT)])
        def red(v, carry):
            acc16 = deg_v[pl.ds(v * 16, 16)]
            for t in range(1, 16):
                acc16 = acc16 + deg_v[pl.ds(t * RPT + v * 16, 16)]
            deg_v[pl.ds(v * 16, 16)] = acc16
            return carry

        lax.fori_loop(0, RPT // 16, red, 0)
        pltpu.sync_copy(deg_v.at[pl.ds(0, RPT)],
                        deg_hbm.at[cid, pl.ds(row0, RPT)])

        out0 = cid * NPAD + row0

        def flush(j, carry):
            pltpu.sync_copy(acc_sh.at[pl.ds(row0 + j * CHUNK, CHUNK)],
                            rows[0])
            pltpu.sync_copy(rows[0],
                            out_hbm.at[pl.ds(out0 + j * CHUNK, CHUNK)])
            return carry

        lax.fori_loop(0, ZCH, flush, 0)

    return sc_agg(table, src_all, dst_all)


def _tc_combine(acc, d0, d1, x, W_rel0, W_rel1, W_loop, b_loop):
    blk = 512
    off = NPAD // blk

    def body(a0, a1, dr0, dr1, xr, w0, w1, wl, br, o):
        agg0 = a0[...] / jnp.maximum(dr0[...], 1.0)
        agg1 = a1[...] / jnp.maximum(dr1[...], 1.0)
        h = jnp.dot(agg0, w0[...], preferred_element_type=jnp.float32)
        h = h + jnp.dot(agg1, w1[...], preferred_element_type=jnp.float32)
        h = h + lax.dot_general(xr[...], wl[...], (((1,), (1,)), ((), ())),
                                preferred_element_type=jnp.float32)
        o[...] = h + br[...]

    return pl.pallas_call(
        body,
        grid=(pl.cdiv(N, blk),),
        in_specs=[
            pl.BlockSpec((blk, D), lambda i: (i, 0)),
            pl.BlockSpec((blk, D), lambda i: (i + off, 0)),
            pl.BlockSpec((blk, 1), lambda i: (i, 0)),
            pl.BlockSpec((blk, 1), lambda i: (i, 0)),
            pl.BlockSpec((blk, D), lambda i: (i, 0)),
            pl.BlockSpec((D, D), lambda i: (0, 0)),
            pl.BlockSpec((D, D), lambda i: (0, 0)),
            pl.BlockSpec((D, D), lambda i: (0, 0)),
            pl.BlockSpec((1, D), lambda i: (0, 0)),
        ],
        out_specs=pl.BlockSpec((blk, D), lambda i: (i, 0)),
        out_shape=jax.ShapeDtypeStruct((N, D), jnp.float32),
    )(acc, acc, d0, d1, x, W_rel0, W_rel1, W_loop, b_loop.reshape(1, D))


def kernel(x, edge_index_rel0, edge_index_rel1, W_rel0, W_rel1, W_loop,
           b_loop):
    table = jnp.pad(x, ((0, NPAD - N), (0, 0)))

    def prep(ei):
        src = jnp.concatenate([ei[0], jnp.zeros((NEP - E,), jnp.int32)])
        dst = jnp.concatenate([ei[1], jnp.full((NEP - E,), N, jnp.int32)])
        return (src.reshape(16, NGRP, G, CHUNK),
                dst.reshape(16, NGRP, G, CHUNK))

    s0, d0 = prep(edge_index_rel0)
    s1, d1 = prep(edge_index_rel1)
    src_all = jnp.concatenate([s0, s1]).astype(jnp.int32)
    dst_all = jnp.concatenate([d0, d1]).astype(jnp.int32)

    acc, deg = _sc_aggregate(table, src_all, dst_all)
    dg0 = deg[0, :N].reshape(N, 1)
    dg1 = deg[1, :N].reshape(N, 1)
    return _tc_combine(acc, dg0, dg1, x, W_rel0, W_rel1, W_loop, b_loop)


# speedup vs baseline: 1.3054x; 1.3054x over previous
import functools

import jax
import jax.numpy as jnp
from jax import lax
from jax.experimental import pallas as pl
from jax.experimental.pallas import tpu as pltpu
from jax.experimental.pallas import tpu_sc as plsc

N = 10000
D = 128
E = 160000

NPAD = 10240
CHUNK = 128
NCH = 79
EPT = NCH * CHUNK
NEP = 16 * EPT
RPT = NPAD // 16
ZCH = RPT // CHUNK
LPC = CHUNK // 16


def _sc_aggregate(table, src_all, dst_all):
    mesh = plsc.VectorSubcoreMesh(core_axis_name="c", subcore_axis_name="s")

    @functools.partial(
        pl.kernel,
        mesh=mesh,
        compiler_params=pltpu.CompilerParams(use_tc_tiling_on_sc=False,
                                             needs_layout_passes=False),
        out_type=(jax.ShapeDtypeStruct((2 * NPAD, D), jnp.float32),
                  jax.ShapeDtypeStruct((2, NPAD), jnp.float32)),
        scratch_types=[
            pltpu.VMEM((NCH, CHUNK), jnp.int32),
            pltpu.VMEM((NCH, CHUNK), jnp.int32),
            pltpu.VMEM((CHUNK, D), jnp.float32),
            pltpu.VMEM((NPAD,), jnp.float32),
            pltpu.VMEM_SHARED((NPAD, D), jnp.float32),
            pltpu.HBM((2, 16, NPAD), jnp.float32),
            pltpu.SemaphoreType.DMA,
        ],
    )
    def sc_agg(table_hbm, src_hbm, dst_hbm, out_hbm, deg_hbm, src_v, dst_v,
               rows_v, deg_v, acc_sh, part_hbm, sem):
        cid = lax.axis_index("c")
        sid = lax.axis_index("s")
        widx = cid * 16 + sid
        row0 = sid * RPT

        zeros16 = jnp.zeros((16,), jnp.float32)
        ones16 = jnp.ones((16,), jnp.float32)

        def zero_row(i, carry):
            for c in range(D // 16):
                rows_v[i, pl.ds(c * 16, 16)] = zeros16
            return carry

        lax.fori_loop(0, CHUNK, zero_row, 0)

        def zero_deg(i, carry):
            deg_v[pl.ds(i * 16, 16)] = zeros16
            return carry

        lax.fori_loop(0, NPAD // 16, zero_deg, 0)
        for j in range(ZCH):
            pltpu.sync_copy(rows_v, acc_sh.at[pl.ds(row0 + j * CHUNK, CHUNK)])
        plsc.subcore_barrier()

        pltpu.sync_copy(src_hbm.at[widx], src_v)
        pltpu.sync_copy(dst_hbm.at[widx], dst_v)

        def body(j, carry):
            pltpu.async_copy(table_hbm.at[src_v.at[j]], rows_v, sem).wait()
            pltpu.sync_copy(rows_v, acc_sh.at[dst_v.at[j]], add=True)
            for l in range(LPC):
                dvec = dst_v[j, pl.ds(l * 16, 16)]
                plsc.addupdate_scatter(deg_v, [dvec], ones16)
            return carry

        lax.fori_loop(0, NCH, body, 0)

        pltpu.sync_copy(deg_v, part_hbm.at[cid, sid])
        plsc.subcore_barrier()
        for t in range(16):
            pltpu.sync_copy(part_hbm.at[cid, t, pl.ds(row0, RPT)],
                            deg_v.at[pl.ds(t * RPT, RPT)])
        def red(v, carry):
            acc16 = deg_v[pl.ds(v * 16, 16)]
            for t in range(1, 16):
                acc16 = acc16 + deg_v[pl.ds(t * RPT + v * 16, 16)]
            deg_v[pl.ds(v * 16, 16)] = acc16
            return carry

        lax.fori_loop(0, RPT // 16, red, 0)
        pltpu.sync_copy(deg_v.at[pl.ds(0, RPT)],
                        deg_hbm.at[cid, pl.ds(row0, RPT)])

        out0 = cid * NPAD + row0

        def flush(j, carry):
            pltpu.sync_copy(acc_sh.at[pl.ds(row0 + j * CHUNK, CHUNK)],
                            out_hbm.at[pl.ds(out0 + j * CHUNK, CHUNK)])
            return carry

        lax.fori_loop(0, ZCH, flush, 0)

    return sc_agg(table, src_all, dst_all)


def _tc_combine(acc, d0, d1, x, W_rel0, W_rel1, W_loop, b_loop):
    blk = 512
    off = NPAD // blk

    def body(a0, a1, dr0, dr1, xr, w0, w1, wl, br, o):
        agg0 = a0[...] / jnp.maximum(dr0[...], 1.0)
        agg1 = a1[...] / jnp.maximum(dr1[...], 1.0)
        h = jnp.dot(agg0, w0[...], preferred_element_type=jnp.float32)
        h = h + jnp.dot(agg1, w1[...], preferred_element_type=jnp.float32)
        h = h + lax.dot_general(xr[...], wl[...], (((1,), (1,)), ((), ())),
                                preferred_element_type=jnp.float32)
        o[...] = h + br[...]

    return pl.pallas_call(
        body,
        grid=(pl.cdiv(N, blk),),
        in_specs=[
            pl.BlockSpec((blk, D), lambda i: (i, 0)),
            pl.BlockSpec((blk, D), lambda i: (i + off, 0)),
            pl.BlockSpec((blk, 1), lambda i: (i, 0)),
            pl.BlockSpec((blk, 1), lambda i: (i, 0)),
            pl.BlockSpec((blk, D), lambda i: (i, 0)),
            pl.BlockSpec((D, D), lambda i: (0, 0)),
            pl.BlockSpec((D, D), lambda i: (0, 0)),
            pl.BlockSpec((D, D), lambda i: (0, 0)),
            pl.BlockSpec((1, D), lambda i: (0, 0)),
        ],
        out_specs=pl.BlockSpec((blk, D), lambda i: (i, 0)),
        out_shape=jax.ShapeDtypeStruct((N, D), jnp.float32),
    )(acc, acc, d0, d1, x, W_rel0, W_rel1, W_loop, b_loop.reshape(1, D))


def kernel(x, edge_index_rel0, edge_index_rel1, W_rel0, W_rel1, W_loop,
           b_loop):
    table = x

    def prep(ei):
        src = jnp.concatenate([ei[0], jnp.zeros((NEP - E,), jnp.int32)])
        dst = jnp.concatenate([ei[1], jnp.full((NEP - E,), N, jnp.int32)])
        return src.reshape(16, NCH, CHUNK), dst.reshape(16, NCH, CHUNK)

    s0, d0 = prep(edge_index_rel0)
    s1, d1 = prep(edge_index_rel1)
    src_all = jnp.concatenate([s0, s1]).astype(jnp.int32)
    dst_all = jnp.concatenate([d0, d1]).astype(jnp.int32)

    acc, deg = _sc_aggregate(table, src_all, dst_all)
    dg0 = deg[0, :N].reshape(N, 1)
    dg1 = deg[1, :N].reshape(N, 1)
    return _tc_combine(acc, dg0, dg1, x, W_rel0, W_rel1, W_loop, b_loop)
